# baseline (speedup 1.0000x reference)
import functools

import jax
import jax.numpy as jnp
from jax import lax
from jax.experimental import pallas as pl
from jax.experimental.pallas import tpu as pltpu
from jax.experimental.pallas import tpu_sc as plsc

NC = 2
NS = 16
NW = NC * NS
C = 128
NBUF = 2
IB = 8
SPLIT = 2


def _sc_degree(col_r, ones128, zeros128, n_acc, K):
    mesh = plsc.VectorSubcoreMesh(core_axis_name="c", subcore_axis_name="s")
    rpt = n_acc // NS

    @functools.partial(
        pl.kernel,
        out_type=jax.ShapeDtypeStruct((NC * n_acc, 128), jnp.float32),
        mesh=mesh,
        scratch_types=[
            pltpu.VMEM((K, C), jnp.int32),
            pltpu.VMEM((C, 128), jnp.float32),
            pltpu.VMEM((C, 128), jnp.float32),
            pltpu.VMEM_SHARED((n_acc, 128), jnp.float32),
            pltpu.SemaphoreType.DMA,
        ],
    )
    def deg_kernel(col_hbm, ones_hbm, zeros_hbm, out_hbm, cidx, ones_v, zeros_v,
                   acc, sem):
        c = lax.axis_index("c")
        s = lax.axis_index("s")
        wid = c * NS + s
        pltpu.sync_copy(col_hbm.at[pl.ds(wid * K, K)], cidx)
        pltpu.sync_copy(ones_hbm, ones_v)
        pltpu.sync_copy(zeros_hbm, zeros_v)
        base = s * rpt
        off = 0
        while off < rpt:
            m = min(C, rpt - off)
            pltpu.sync_copy(zeros_v.at[pl.ds(0, m)], acc.at[pl.ds(base + off, m)])
            off += m
        plsc.subcore_barrier()

        descs = [pltpu.async_copy(ones_v, acc.at[cidx.at[k]], sem, add=True)
                 for k in range(K)]
        for d in descs:
            d.wait()
        plsc.subcore_barrier()
        pltpu.sync_copy(acc.at[pl.ds(base, rpt)],
                        out_hbm.at[pl.ds(c * n_acc + base, rpt)])

    return deg_kernel(col_r, ones128, zeros128)


def _sc_aggregate(y, row_r, col_r, zeros128, n_acc, K, n, dtype):
    mesh = plsc.VectorSubcoreMesh(core_axis_name="c", subcore_axis_name="s")
    rpt = n_acc // NS

    nblk = K // IB

    @functools.partial(
        pl.kernel,
        out_type=jax.ShapeDtypeStruct((NC * n_acc, 128), dtype),
        mesh=mesh,
        scratch_types=[
            [pltpu.VMEM((IB, C), jnp.int32)] * 2,
            [pltpu.VMEM((IB, C), jnp.int32)] * 2,
            [pltpu.VMEM((C, 128), dtype)] * NBUF,
            pltpu.VMEM_SHARED((n_acc, 128), dtype),
            [pltpu.SemaphoreType.DMA] * (NBUF * SPLIT),
            [pltpu.SemaphoreType.DMA] * (NBUF * SPLIT),
            [pltpu.SemaphoreType.DMA] * 4,
        ],
    )
    def agg_kernel(y_hbm, row_hbm, col_hbm, z_hbm, out_hbm,
                   ridx, cidx, bufs, acc, gsems, ssems, isems):
        c = lax.axis_index("c")
        s = lax.axis_index("s")
        wid = c * NS + s
        tbase = wid * K
        pltpu.sync_copy(z_hbm, bufs[0])
        base = s * rpt
        off = 0
        while off < rpt:
            m = min(C, rpt - off)
            pltpu.sync_copy(bufs[0].at[pl.ds(0, m)], acc.at[pl.ds(base + off, m)])
            off += m

        def idx_prefetch(j):
            slot = j % 2
            return (
                pltpu.async_copy(row_hbm.at[pl.ds(tbase + j * IB, IB)],
                                 ridx[slot], isems[2 * slot]),
                pltpu.async_copy(col_hbm.at[pl.ds(tbase + j * IB, IB)],
                                 cidx[slot], isems[2 * slot + 1]),
            )

        pltpu.sync_copy(row_hbm.at[pl.ds(tbase, IB)], ridx[0])
        pltpu.sync_copy(col_hbm.at[pl.ds(tbase, IB)], cidx[0])
        idx_d = [None, None]
        if nblk > 1:
            idx_d[1] = idx_prefetch(1)
        plsc.subcore_barrier()

        gd = [None] * NBUF
        sd = [None] * NBUF
        issued_blk = 0

        CS = C // SPLIT

        def gather(g, b):
            nonlocal issued_blk
            j = g // IB
            if j > issued_blk:
                for d in idx_d[j % 2]:
                    d.wait()
                issued_blk = j
            gd[b] = [
                pltpu.async_copy(
                    y_hbm.at[ridx[j % 2].at[g % IB, pl.ds(h * CS, CS)]],
                    bufs[b].at[pl.ds(h * CS, CS)], gsems[b * SPLIT + h])
                for h in range(SPLIT)
            ]

        for k in range(min(NBUF, K)):
            gather(k, k)
        for k in range(K):
            b = k % NBUF
            j = k // IB
            if k % IB == NBUF and j + 1 < nblk and j >= 1:
                idx_d[(j + 1) % 2] = idx_prefetch(j + 1)
            for d in gd[b]:
                d.wait()
            sd[b] = [
                pltpu.async_copy(
                    bufs[b].at[pl.ds(h * CS, CS)],
                    acc.at[cidx[j % 2].at[k % IB, pl.ds(h * CS, CS)]],
                    ssems[b * SPLIT + h], add=True)
                for h in range(SPLIT)
            ]
            if k + NBUF < K:
                for d in sd[b]:
                    d.wait()
                gather(k + NBUF, b)
        for k in range(max(0, K - NBUF), K):
            for d in sd[k % NBUF]:
                d.wait()
        plsc.subcore_barrier()
        pltpu.sync_copy(acc.at[pl.ds(base, rpt)],
                        out_hbm.at[pl.ds(c * n_acc + base, rpt)])

    return agg_kernel(y, row_r, col_r, zeros128)


def _tc_matmul(x, W):
    def body(x_ref, w_ref, o_ref):
        o_ref[...] = jnp.dot(x_ref[...], w_ref[...],
                             preferred_element_type=jnp.float32)

    return pl.pallas_call(
        body,
        out_shape=jax.ShapeDtypeStruct((x.shape[0], 128), jnp.float32),
    )(x, W)


def _tc_scale(xw, deg_p, n, n_acc):
    def body(xw_ref, d_ref, y_ref, yb_ref):
        d = d_ref[...]
        deg = d[0:n, 0:1] + d[n_acc:n_acc + n, 0:1] + 1.0
        y = xw_ref[...] * lax.rsqrt(deg)
        y_ref[...] = y
        yb_ref[...] = y.astype(jnp.bfloat16)

    return pl.pallas_call(
        body,
        out_shape=(jax.ShapeDtypeStruct((n, 128), jnp.float32),
                   jax.ShapeDtypeStruct((n, 128), jnp.bfloat16)),
    )(xw, deg_p)


def _tc_final(S, y, deg_p, b, n, n_acc):
    def body(s_ref, y_ref, d_ref, b_ref, o_ref):
        d = d_ref[...]
        deg = d[0:n, 0:1] + d[n_acc:n_acc + n, 0:1] + 1.0
        s = s_ref[...].astype(jnp.float32)
        agg = s[0:n, :] + s[n_acc:n_acc + n, :] + y_ref[...]
        o_ref[...] = jnp.maximum(agg * lax.rsqrt(deg) + b_ref[...], 0.0)

    return pl.pallas_call(
        body,
        out_shape=jax.ShapeDtypeStruct((n, 128), jnp.float32),
    )(S, y, deg_p, b.reshape(1, 128))


def kernel(x, edge_index, W, b):
    n = x.shape[0]
    e = edge_index.shape[1]
    K = -(-(-(-e // (NW * C))) // 8) * 8
    e_pad = NW * K * C
    row = edge_index[0]
    col = edge_index[1]
    if e_pad > e:
        row = jnp.concatenate([row, jnp.zeros((e_pad - e,), jnp.int32)])
        col = jnp.concatenate([col, jnp.full((e_pad - e,), n, jnp.int32)])
    row_r = row.reshape(NW * K, C)
    col_r = col.reshape(NW * K, C)
    n_acc = -(-(n + 1) // (NS * 8)) * (NS * 8)

    ones128 = jnp.ones((C, 128), jnp.float32)
    zeros128 = jnp.zeros((C, 128), jnp.float32)
    zeros128b = jnp.zeros((C, 128), jnp.bfloat16)

    xw = _tc_matmul(x, W)
    deg_p = _sc_degree(col_r, ones128, zeros128, n_acc, K)
    y, _ = _tc_scale(xw, deg_p, n, n_acc)
    S = _sc_aggregate(y, row_r, col_r, zeros128, n_acc, K, n, jnp.float32)
    return _tc_final(S, y, deg_p, b, n, n_acc)


# speedup vs baseline: 1.3237x; 1.3237x over previous
import functools

import jax
import jax.numpy as jnp
from jax import lax
from jax.experimental import pallas as pl
from jax.experimental.pallas import tpu as pltpu
from jax.experimental.pallas import tpu_sc as plsc

NC = 2
NS = 16
NW = NC * NS
C = 128
NBUF = 2
IB = 8
SPLIT = 2


def _sc_degree(col_r, ones128, zeros128, n_acc, K):
    mesh = plsc.VectorSubcoreMesh(core_axis_name="c", subcore_axis_name="s")
    rpt = n_acc // NS

    @functools.partial(
        pl.kernel,
        out_type=jax.ShapeDtypeStruct((NC * n_acc, 128), jnp.float32),
        mesh=mesh,
        scratch_types=[
            pltpu.VMEM((K, C), jnp.int32),
            pltpu.VMEM((C, 128), jnp.float32),
            pltpu.VMEM((C, 128), jnp.float32),
            pltpu.VMEM_SHARED((n_acc, 128), jnp.float32),
            pltpu.SemaphoreType.DMA,
        ],
    )
    def deg_kernel(col_hbm, ones_hbm, zeros_hbm, out_hbm, cidx, ones_v, zeros_v,
                   acc, sem):
        c = lax.axis_index("c")
        s = lax.axis_index("s")
        wid = c * NS + s
        pltpu.sync_copy(col_hbm.at[pl.ds(wid * K, K)], cidx)
        pltpu.sync_copy(ones_hbm, ones_v)
        pltpu.sync_copy(zeros_hbm, zeros_v)
        base = s * rpt
        off = 0
        while off < rpt:
            m = min(C, rpt - off)
            pltpu.sync_copy(zeros_v.at[pl.ds(0, m)], acc.at[pl.ds(base + off, m)])
            off += m
        plsc.subcore_barrier()

        descs = [pltpu.async_copy(ones_v, acc.at[cidx.at[k]], sem, add=True)
                 for k in range(K)]
        for d in descs:
            d.wait()
        plsc.subcore_barrier()
        pltpu.sync_copy(acc.at[pl.ds(base, rpt)],
                        out_hbm.at[pl.ds(c * n_acc + base, rpt)])

    return deg_kernel(col_r, ones128, zeros128)


def _sc_aggregate(y, row_r, col_r, zeros128, n_acc, K, n, dtype):
    mesh = plsc.VectorSubcoreMesh(core_axis_name="c", subcore_axis_name="s")
    rpt = n_acc // NS

    nblk = K // IB

    @functools.partial(
        pl.kernel,
        out_type=jax.ShapeDtypeStruct((NC * n_acc, 128), dtype),
        mesh=mesh,
        scratch_types=[
            [pltpu.VMEM((IB, C), jnp.int32)] * 2,
            [pltpu.VMEM((IB, C), jnp.int32)] * 2,
            [pltpu.VMEM((C, 128), dtype)] * NBUF,
            pltpu.VMEM_SHARED((n_acc, 128), dtype),
            [pltpu.SemaphoreType.DMA] * (NBUF * SPLIT),
            [pltpu.SemaphoreType.DMA] * (NBUF * SPLIT),
            [pltpu.SemaphoreType.DMA] * 4,
        ],
    )
    def agg_kernel(y_hbm, row_hbm, col_hbm, z_hbm, out_hbm,
                   ridx, cidx, bufs, acc, gsems, ssems, isems):
        c = lax.axis_index("c")
        s = lax.axis_index("s")
        wid = c * NS + s
        tbase = wid * K
        pltpu.sync_copy(z_hbm, bufs[0])
        base = s * rpt
        off = 0
        while off < rpt:
            m = min(C, rpt - off)
            pltpu.sync_copy(bufs[0].at[pl.ds(0, m)], acc.at[pl.ds(base + off, m)])
            off += m

        def idx_prefetch(j):
            slot = j % 2
            return (
                pltpu.async_copy(row_hbm.at[pl.ds(tbase + j * IB, IB)],
                                 ridx[slot], isems[2 * slot]),
                pltpu.async_copy(col_hbm.at[pl.ds(tbase + j * IB, IB)],
                                 cidx[slot], isems[2 * slot + 1]),
            )

        pltpu.sync_copy(row_hbm.at[pl.ds(tbase, IB)], ridx[0])
        pltpu.sync_copy(col_hbm.at[pl.ds(tbase, IB)], cidx[0])
        idx_d = [None, None]
        if nblk > 1:
            idx_d[1] = idx_prefetch(1)
        plsc.subcore_barrier()

        gd = [None] * NBUF
        sd = [None] * NBUF
        issued_blk = 0

        CS = C // SPLIT

        def gather(g, b):
            nonlocal issued_blk
            j = g // IB
            if j > issued_blk:
                for d in idx_d[j % 2]:
                    d.wait()
                issued_blk = j
            gd[b] = [
                pltpu.async_copy(
                    y_hbm.at[ridx[j % 2].at[g % IB, pl.ds(h * CS, CS)]],
                    bufs[b].at[pl.ds(h * CS, CS)], gsems[b * SPLIT + h])
                for h in range(SPLIT)
            ]

        for k in range(min(NBUF, K)):
            gather(k, k)
        for k in range(K):
            b = k % NBUF
            j = k // IB
            if k % IB == NBUF and j + 1 < nblk and j >= 1:
                idx_d[(j + 1) % 2] = idx_prefetch(j + 1)
            for d in gd[b]:
                d.wait()
            sd[b] = [
                pltpu.async_copy(
                    bufs[b].at[pl.ds(h * CS, CS)],
                    acc.at[cidx[j % 2].at[k % IB, pl.ds(h * CS, CS)]],
                    ssems[b * SPLIT + h], add=True)
                for h in range(SPLIT)
            ]
            if k + NBUF < K:
                for d in sd[b]:
                    d.wait()
                gather(k + NBUF, b)
        for k in range(max(0, K - NBUF), K):
            for d in sd[k % NBUF]:
                d.wait()
        plsc.subcore_barrier()
        pltpu.sync_copy(acc.at[pl.ds(base, rpt)],
                        out_hbm.at[pl.ds(c * n_acc + base, rpt)])

    return agg_kernel(y, row_r, col_r, zeros128)


def _tc_matmul(x, W):
    def body(x_ref, w_ref, o_ref):
        o_ref[...] = jnp.dot(x_ref[...], w_ref[...],
                             preferred_element_type=jnp.float32)

    return pl.pallas_call(
        body,
        out_shape=jax.ShapeDtypeStruct((x.shape[0], 128), jnp.float32),
    )(x, W)


def _tc_scale(xw, deg_p, n, n_acc):
    def body(xw_ref, d_ref, y_ref):
        d = d_ref[...]
        deg = d[0:n, 0:1] + d[n_acc:n_acc + n, 0:1] + 1.0
        y_ref[...] = xw_ref[...] * lax.rsqrt(deg)

    return pl.pallas_call(
        body,
        out_shape=jax.ShapeDtypeStruct((n, 128), jnp.float32),
    )(xw, deg_p)


def _tc_final(S, y, deg_p, b, n, n_acc):
    def body(s_ref, y_ref, d_ref, b_ref, o_ref):
        d = d_ref[...]
        deg = d[0:n, 0:1] + d[n_acc:n_acc + n, 0:1] + 1.0
        s = s_ref[...].astype(jnp.float32)
        agg = s[0:n, :] + s[n_acc:n_acc + n, :] + y_ref[...]
        o_ref[...] = jnp.maximum(agg * lax.rsqrt(deg) + b_ref[...], 0.0)

    return pl.pallas_call(
        body,
        out_shape=jax.ShapeDtypeStruct((n, 128), jnp.float32),
    )(S, y, deg_p, b.reshape(1, 128))


def kernel(x, edge_index, W, b):
    n = x.shape[0]
    e = edge_index.shape[1]
    K = -(-(-(-e // (NW * C))) // 8) * 8
    e_pad = NW * K * C
    row = edge_index[0]
    col = edge_index[1]
    if e_pad > e:
        row = jnp.concatenate([row, jnp.zeros((e_pad - e,), jnp.int32)])
        col = jnp.concatenate([col, jnp.full((e_pad - e,), n, jnp.int32)])
    row_r = row.reshape(NW * K, C)
    col_r = col.reshape(NW * K, C)
    n_acc = -(-(n + 1) // (NS * 8)) * (NS * 8)

    ones128 = jnp.ones((C, 128), jnp.float32)
    zeros128 = jnp.zeros((C, 128), jnp.float32)
    zeros128b = jnp.zeros((C, 128), jnp.bfloat16)

    xw = _tc_matmul(x, W)
    deg_p = _sc_degree(col_r, ones128, zeros128, n_acc, K)
    y = _tc_scale(xw, deg_p, n, n_acc)
    S = _sc_aggregate(y, row_r, col_r, zeros128, n_acc, K, n, jnp.float32)
    return _tc_final(S, y, deg_p, b, n, n_acc)
